# Initial kernel scaffold; baseline (speedup 1.0000x reference)
import jax
import jax.numpy as jnp
from jax.experimental import pallas as pl


def kernel(h, W, norm, edge_index):
    raise NotImplementedError("write your pallas kernel here")


# speedup vs baseline: 4.6058x; 4.6058x over previous
import functools

import jax
import jax.numpy as jnp
from jax import lax
from jax.experimental import pallas as pl
from jax.experimental.pallas import tpu as pltpu
from jax.experimental.pallas import tpu_sc as plsc

NC = 2
NS = 16
NW = NC * NS
CHUNK = 128


def _matmul_scale_kernel(h_ref, w_ref, n_ref, o_ref):
  o_ref[...] = jnp.dot(h_ref[...], w_ref[...],
                       preferred_element_type=jnp.float32) * n_ref[...]


def _combine_kernel(p_ref, n_ref, o_ref, *, square):
  s = n_ref[...]
  if square:
    s = s * s
  o_ref[...] = (p_ref[0] + p_ref[1]) * s


def _make_hop(n, n_pad, ch, d):
  rows_per_tile = n_pad // NS
  mesh = plsc.VectorSubcoreMesh(core_axis_name="c", subcore_axis_name="s")

  @functools.partial(
      pl.kernel,
      mesh=mesh,
      out_type=jax.ShapeDtypeStruct((NC, n_pad, d), jnp.float32),
      scratch_types=[
          pltpu.VMEM((ch, CHUNK), jnp.int32),
          pltpu.VMEM((ch, CHUNK), jnp.int32),
          pltpu.VMEM((CHUNK, d), jnp.float32),
          pltpu.VMEM_SHARED((n_pad, d), jnp.float32),
          pltpu.SemaphoreType.DMA,
      ],
  )
  def hop(x_hbm, src_hbm, dst_hbm, z_hbm, out_hbm,
          src_v, dst_v, rows_v, acc_sh, sem):
    c = lax.axis_index("c")
    s = lax.axis_index("s")
    w = c * NS + s

    pltpu.sync_copy(z_hbm, acc_sh.at[pl.ds(s * rows_per_tile, rows_per_tile)])
    pltpu.sync_copy(src_hbm.at[w], src_v)
    pltpu.sync_copy(dst_hbm.at[w], dst_v)
    plsc.subcore_barrier()

    def body(j, carry):
      pltpu.async_copy(x_hbm.at[src_v.at[j]], rows_v, sem).wait()
      pltpu.sync_copy(rows_v, acc_sh.at[dst_v.at[j]], add=True)
      return carry

    lax.fori_loop(0, ch, body, 0, unroll=False)

    plsc.subcore_barrier()
    pltpu.sync_copy(acc_sh.at[pl.ds(s * rows_per_tile, rows_per_tile)],
                    out_hbm.at[c, pl.ds(s * rows_per_tile, rows_per_tile)])

  return hop


def kernel(h, W, norm, edge_index):
  n, d_in = h.shape
  d = W.shape[1]
  e = edge_index.shape[1]

  epw = -(-e // NW)
  ch = -(-epw // CHUNK)
  e_pad = NW * ch * CHUNK
  n_pad = -(-(n + 1) // (NS * 8)) * (NS * 8)
  rows_per_tile = n_pad // NS

  src = jnp.concatenate(
      [edge_index[0], jnp.zeros((e_pad - e,), jnp.int32)]).reshape(NW, ch, CHUNK)
  dst = jnp.concatenate(
      [edge_index[1], jnp.full((e_pad - e,), n, jnp.int32)]).reshape(NW, ch, CHUNK)
  z = jnp.zeros((rows_per_tile, d), jnp.float32)

  rb = 1000
  matmul_scale = pl.pallas_call(
      _matmul_scale_kernel,
      grid=(n // rb,),
      in_specs=[
          pl.BlockSpec((rb, d_in), lambda i: (i, 0)),
          pl.BlockSpec((d_in, d), lambda i: (0, 0)),
          pl.BlockSpec((rb, 1), lambda i: (i, 0)),
      ],
      out_specs=pl.BlockSpec((rb, d), lambda i: (i, 0)),
      out_shape=jax.ShapeDtypeStruct((n, d), jnp.float32),
  )

  def combine(square):
    return pl.pallas_call(
        functools.partial(_combine_kernel, square=square),
        grid=(n // rb,),
        in_specs=[
            pl.BlockSpec((NC, rb, d), lambda i: (0, i, 0)),
            pl.BlockSpec((rb, 1), lambda i: (i, 0)),
        ],
        out_specs=pl.BlockSpec((rb, d), lambda i: (i, 0)),
        out_shape=jax.ShapeDtypeStruct((n, d), jnp.float32),
    )

  hop = _make_hop(n, n_pad, ch, d)

  x = matmul_scale(h, W, norm)
  p = hop(x, src, dst, z)
  x = combine(square=True)(p, norm)
  p = hop(x, src, dst, z)
  return combine(square=False)(p, norm)
